# baseline (speedup 1.0000x reference)
import functools

import jax
import jax.numpy as jnp
from jax import lax
from jax.experimental import pallas as pl
from jax.experimental.pallas import tpu as pltpu
from jax.experimental.pallas import tpu_sc as plsc

B = 16384
D = 32
L = 16
NC = 2
NS = 16
NW = NC * NS
BPW = B // NW
CHUNK = 128
NCHUNK = BPW // CHUNK
GROUPS = BPW // L


def _make_sc_kernel():
    mesh = plsc.VectorSubcoreMesh(core_axis_name="c", subcore_axis_name="s")

    @functools.partial(
        pl.kernel,
        mesh=mesh,
        compiler_params=pltpu.CompilerParams(
            needs_layout_passes=False, use_tc_tiling_on_sc=False),
        out_type=(
            jax.ShapeDtypeStruct((B,), jnp.float32),
            jax.ShapeDtypeStruct((B,), jnp.float32),
        ),
        scratch_types=[
            pltpu.VMEM((BPW,), jnp.int32),
            pltpu.VMEM((BPW,), jnp.int32),
            pltpu.VMEM((BPW,), jnp.int32),
            pltpu.VMEM((BPW, D), jnp.float32),
            pltpu.VMEM((BPW, D), jnp.float32),
            pltpu.VMEM((BPW, D), jnp.float32),
            pltpu.VMEM((BPW,), jnp.float32),
            pltpu.VMEM((BPW,), jnp.float32),
            pltpu.VMEM((BPW,), jnp.float32),
            pltpu.VMEM((BPW,), jnp.float32),
            pltpu.VMEM((BPW,), jnp.float32),
            pltpu.VMEM((L,), jnp.float32),
            pltpu.SemaphoreType.DMA,
        ],
    )
    def k(user_hbm, item_i_hbm, item_j_hbm, eu_hbm, ei_hbm, bu_hbm, bi_hbm,
          gb_hbm, out_i_hbm, out_j_hbm,
          idx_u, idx_i, idx_j, eu_v, ei_iv, ei_jv, bu_v, bi_iv, bi_jv,
          out_iv, out_jv, gb_v, sem):
        wid = lax.axis_index("s") * NC + lax.axis_index("c")
        base = wid * BPW

        pltpu.sync_copy(user_hbm.at[pl.ds(base, BPW)], idx_u)
        pltpu.sync_copy(item_i_hbm.at[pl.ds(base, BPW)], idx_i)
        pltpu.sync_copy(item_j_hbm.at[pl.ds(base, BPW)], idx_j)
        pltpu.sync_copy(gb_hbm, gb_v)

        copies = []
        for c in range(NCHUNK):
            s = pl.ds(c * CHUNK, CHUNK)
            copies.append(pltpu.async_copy(eu_hbm.at[idx_u.at[s]], eu_v.at[s], sem))
            copies.append(pltpu.async_copy(ei_hbm.at[idx_i.at[s]], ei_iv.at[s], sem))
            copies.append(pltpu.async_copy(ei_hbm.at[idx_j.at[s]], ei_jv.at[s], sem))
            copies.append(pltpu.async_copy(bu_hbm.at[idx_u.at[s]], bu_v.at[s], sem))
            copies.append(pltpu.async_copy(bi_hbm.at[idx_i.at[s]], bi_iv.at[s], sem))
            copies.append(pltpu.async_copy(bi_hbm.at[idx_j.at[s]], bi_jv.at[s], sem))
        for cp in copies:
            cp.wait()

        gb = gb_v[...]

        def group_body(g, carry):
            bidx = g * L + lax.iota(jnp.int32, L)
            bs = pl.ds(g * L, L)
            bu16 = bu_v[bs]
            bii16 = bi_iv[bs]
            bij16 = bi_jv[bs]
            acc_i = gb + bu16 + bii16
            acc_j = gb + bu16 + bij16
            for d in range(D):
                dd = jnp.full((L,), d, jnp.int32)
                u = plsc.load_gather(eu_v, [bidx, dd])
                vi = plsc.load_gather(ei_iv, [bidx, dd])
                vj = plsc.load_gather(ei_jv, [bidx, dd])
                acc_i = acc_i + u * vi
                acc_j = acc_j + u * vj
            out_iv[pl.ds(g * L, L)] = acc_i
            out_jv[pl.ds(g * L, L)] = acc_j
            return carry

        lax.fori_loop(0, GROUPS, group_body, 0)

        pltpu.sync_copy(out_iv, out_i_hbm.at[pl.ds(base, BPW)])
        pltpu.sync_copy(out_jv, out_j_hbm.at[pl.ds(base, BPW)])

    return k


_SC_KERNEL = _make_sc_kernel()


def kernel(user, item_i, item_j, embed_user, embed_item, bias_user, bias_item,
           global_bias):
    gb16 = jnp.broadcast_to(global_bias.astype(jnp.float32), (L,))
    return _SC_KERNEL(
        user.astype(jnp.int32),
        item_i.astype(jnp.int32),
        item_j.astype(jnp.int32),
        embed_user,
        embed_item,
        bias_user.reshape(-1),
        bias_item.reshape(-1),
        gb16,
    )
